# baseline (speedup 1.0000x reference)
import functools

import jax
import jax.numpy as jnp
from jax import lax
from jax.experimental import pallas as pl
from jax.experimental.pallas import tpu as pltpu
from jax.experimental.pallas import tpu_sc as plsc

_BIN_SIZE = 4096


def _ffd_bins(lengths, bin_size):
    order = sorted(range(len(lengths)), key=lambda i: lengths[i], reverse=True)
    bins = [[]]
    for idx in order:
        L = lengths[idx]
        placed = False
        for b in bins:
            if sum(lengths[j] for j in b) + L <= bin_size:
                b.append(idx)
                placed = True
                break
        if not placed:
            bins.append([idx])
    return bins


def kernel(seq0, seq1, seq2, seq3, seq4, seq5, seq6, seq7):
    seqs = [seq0, seq1, seq2, seq3, seq4, seq5, seq6, seq7]
    lengths = [int(s.shape[0]) for s in seqs]
    hidden = int(seqs[0].shape[1])
    bins = _ffd_bins(lengths, _BIN_SIZE)
    used = [sum(lengths[j] for j in b) for b in bins]
    max_len = max(used)
    nbins = len(bins)

    copies = []
    pads = []
    for b, members in enumerate(bins):
        off = 0
        for j in members:
            copies.append((b, off, j))
            off += lengths[j]
        if off < max_len:
            pads.append((b, off, max_len - off))

    info = plsc.get_sparse_core_info()
    NC, NS = int(info.num_cores), int(info.num_subcores)
    W = NC * NS

    assert all(L % W == 0 for L in lengths), lengths
    assert all(p % W == 0 and (p // W) % 16 == 0 for (_, _, p) in pads)
    assert max_len % (W * 16) == 0 and hidden % 16 == 0

    km = max_len // W

    mesh = plsc.VectorSubcoreMesh(core_axis_name="c", subcore_axis_name="s")

    @functools.partial(
        pl.kernel,
        mesh=mesh,
        out_type=(
            jax.ShapeDtypeStruct((nbins, max_len, hidden), jnp.float32),
            jax.ShapeDtypeStruct((nbins, max_len), jnp.float32),
        ),
        scratch_types=[
            pltpu.VMEM((16, hidden), jnp.float32),
            pltpu.VMEM((nbins, km), jnp.float32),
            pltpu.SemaphoreType.DMA,
        ],
    )
    def _pack(s0, s1, s2, s3, s4, s5, s6, s7, out_ref, mask_ref, zbuf, mbuf, sem):
        seq_refs = [s0, s1, s2, s3, s4, s5, s6, s7]
        wid = lax.axis_index("s") * NC + lax.axis_index("c")
        handles = []

        for (b, dst0, j) in copies:
            k = lengths[j] // W
            src0 = wid * k
            handles.append(
                pltpu.async_copy(
                    seq_refs[j].at[pl.ds(src0, k), :],
                    out_ref.at[b, pl.ds(dst0 + src0, k), :],
                    sem,
                )
            )

        def _zrow(i, c):
            def _zcol(jj, cc):
                zbuf[i, pl.ds(jj * 16, 16)] = jnp.zeros((16,), jnp.float32)
                return cc
            return lax.fori_loop(0, hidden // 16, _zcol, c)
        lax.fori_loop(0, 16, _zrow, 0)

        for (b, off, p) in pads:
            kp = p // W
            base = off + wid * kp
            for c0 in range(0, kp, 16):
                handles.append(
                    pltpu.async_copy(
                        zbuf, out_ref.at[b, pl.ds(base + c0, 16), :], sem
                    )
                )

        iot = lax.iota(jnp.int32, 16)
        col0 = wid * km
        for b in range(nbins):
            for jj in range(km // 16):
                col = col0 + jj * 16 + iot
                mbuf[b, pl.ds(jj * 16, 16)] = jnp.where(
                    col < used[b], jnp.float32(1.0), jnp.float32(0.0)
                )
            handles.append(
                pltpu.async_copy(
                    mbuf.at[pl.ds(b, 1)],
                    mask_ref.at[pl.ds(b, 1), pl.ds(col0, km)],
                    sem,
                )
            )

        for h in handles:
            h.wait()

    return _pack(*seqs)


# speedup vs baseline: 30.0030x; 30.0030x over previous
import functools

import jax
import jax.numpy as jnp
from jax import lax
from jax.experimental import pallas as pl
from jax.experimental.pallas import tpu as pltpu
from jax.experimental.pallas import tpu_sc as plsc

_BIN_SIZE = 4096
_CHUNK = 48


def _ffd_bins(lengths, bin_size):
    order = sorted(range(len(lengths)), key=lambda i: lengths[i], reverse=True)
    bins = [[]]
    for idx in order:
        L = lengths[idx]
        placed = False
        for b in bins:
            if sum(lengths[j] for j in b) + L <= bin_size:
                b.append(idx)
                placed = True
                break
        if not placed:
            bins.append([idx])
    return bins


def kernel(seq0, seq1, seq2, seq3, seq4, seq5, seq6, seq7):
    seqs = [seq0, seq1, seq2, seq3, seq4, seq5, seq6, seq7]
    lengths = [int(s.shape[0]) for s in seqs]
    hidden = int(seqs[0].shape[1])
    bins = _ffd_bins(lengths, _BIN_SIZE)
    used = [sum(lengths[j] for j in b) for b in bins]
    max_len = max(used)
    nbins = len(bins)

    copies = []
    pads = []
    for b, members in enumerate(bins):
        off = 0
        for j in members:
            copies.append((b, off, j))
            off += lengths[j]
        if off < max_len:
            pads.append((b, off, max_len - off))

    info = plsc.get_sparse_core_info()
    NC, NS = int(info.num_cores), int(info.num_subcores)
    W = NC * NS

    assert all(L % W == 0 for L in lengths), lengths
    assert all(p % W == 0 and (p // W) % 16 == 0 for (_, _, p) in pads)
    assert max_len % (W * 16) == 0 and hidden % 16 == 0

    km = max_len // W

    plan = []
    for (b, dst0, j) in copies:
        share = lengths[j] // W
        for rel in range(0, share, _CHUNK):
            plan.append((j, b, dst0, share, rel, min(_CHUNK, share - rel)))
    nchunks = len(plan)

    mesh = plsc.VectorSubcoreMesh(core_axis_name="c", subcore_axis_name="s")

    @functools.partial(
        pl.kernel,
        mesh=mesh,
        out_type=(
            jax.ShapeDtypeStruct((nbins, max_len, hidden), jnp.float32),
            jax.ShapeDtypeStruct((nbins, max_len), jnp.float32),
        ),
        scratch_types=[
            pltpu.VMEM((_CHUNK, hidden), jnp.float32),
            pltpu.VMEM((_CHUNK, hidden), jnp.float32),
            pltpu.VMEM((16, hidden), jnp.float32),
            pltpu.VMEM((nbins, km), jnp.float32),
            pltpu.SemaphoreType.DMA,
            pltpu.SemaphoreType.DMA,
            pltpu.SemaphoreType.DMA,
            pltpu.SemaphoreType.DMA,
            pltpu.SemaphoreType.DMA,
        ],
    )
    def _pack(s0, s1, s2, s3, s4, s5, s6, s7, out_ref, mask_ref,
              buf0, buf1, zbuf, mbuf, rsem0, rsem1, wsem0, wsem1, zsem):
        seq_refs = [s0, s1, s2, s3, s4, s5, s6, s7]
        bufs = [buf0, buf1]
        rsems = [rsem0, rsem1]
        wsems = [wsem0, wsem1]
        wid = lax.axis_index("s") * NC + lax.axis_index("c")

        rh = [None] * nchunks
        wh = [None] * nchunks

        def start_read(i):
            j, b, dst0, share, rel, cnt = plan[i]
            rh[i] = pltpu.async_copy(
                seq_refs[j].at[pl.ds(wid * share + rel, cnt), :],
                bufs[i % 2].at[pl.ds(0, cnt)],
                rsems[i % 2],
            )

        def start_write(i):
            j, b, dst0, share, rel, cnt = plan[i]
            wh[i] = pltpu.async_copy(
                bufs[i % 2].at[pl.ds(0, cnt)],
                out_ref.at[b, pl.ds(dst0 + wid * share + rel, cnt), :],
                wsems[i % 2],
            )

        start_read(0)

        def _zrow(i, c):
            def _zcol(jj, cc):
                zbuf[i, pl.ds(jj * 16, 16)] = jnp.zeros((16,), jnp.float32)
                return cc
            return lax.fori_loop(0, hidden // 16, _zcol, c)
        lax.fori_loop(0, 16, _zrow, 0)

        aux = []
        for (b, off, p) in pads:
            kp = p // W
            base = off + wid * kp
            for c0 in range(0, kp, 16):
                aux.append(
                    pltpu.async_copy(
                        zbuf, out_ref.at[b, pl.ds(base + c0, 16), :], zsem
                    )
                )

        for i in range(nchunks):
            if i + 1 < nchunks:
                if i >= 1:
                    wh[i - 1].wait()
                start_read(i + 1)
            rh[i].wait()
            start_write(i)

        iot = lax.iota(jnp.int32, 16)
        col0 = wid * km
        for b in range(nbins):
            for jj in range(km // 16):
                col = col0 + jj * 16 + iot
                mbuf[b, pl.ds(jj * 16, 16)] = jnp.where(
                    col < used[b], jnp.float32(1.0), jnp.float32(0.0)
                )
            aux.append(
                pltpu.async_copy(
                    mbuf.at[pl.ds(b, 1)],
                    mask_ref.at[pl.ds(b, 1), pl.ds(col0, km)],
                    zsem,
                )
            )

        wh[nchunks - 1].wait()
        if nchunks >= 2:
            wh[nchunks - 2].wait()
        for h in aux:
            h.wait()

    return _pack(*seqs)


# speedup vs baseline: 31.1224x; 1.0373x over previous
import functools

import jax
import jax.numpy as jnp
from jax import lax
from jax.experimental import pallas as pl
from jax.experimental.pallas import tpu as pltpu
from jax.experimental.pallas import tpu_sc as plsc

_BIN_SIZE = 4096
_CHUNK = 24
_NBUF = 4


def _ffd_bins(lengths, bin_size):
    order = sorted(range(len(lengths)), key=lambda i: lengths[i], reverse=True)
    bins = [[]]
    for idx in order:
        L = lengths[idx]
        placed = False
        for b in bins:
            if sum(lengths[j] for j in b) + L <= bin_size:
                b.append(idx)
                placed = True
                break
        if not placed:
            bins.append([idx])
    return bins


def kernel(seq0, seq1, seq2, seq3, seq4, seq5, seq6, seq7):
    seqs = [seq0, seq1, seq2, seq3, seq4, seq5, seq6, seq7]
    lengths = [int(s.shape[0]) for s in seqs]
    hidden = int(seqs[0].shape[1])
    bins = _ffd_bins(lengths, _BIN_SIZE)
    used = [sum(lengths[j] for j in b) for b in bins]
    max_len = max(used)
    nbins = len(bins)

    copies = []
    pads = []
    for b, members in enumerate(bins):
        off = 0
        for j in members:
            copies.append((b, off, j))
            off += lengths[j]
        if off < max_len:
            pads.append((b, off, max_len - off))

    info = plsc.get_sparse_core_info()
    NC, NS = int(info.num_cores), int(info.num_subcores)
    W = NC * NS

    assert all(L % W == 0 for L in lengths), lengths
    assert all(p % W == 0 and (p // W) % 16 == 0 for (_, _, p) in pads)
    assert max_len % (W * 16) == 0 and hidden % 16 == 0

    km = max_len // W

    plan = []
    for (b, dst0, j) in copies:
        share = lengths[j] // W
        for rel in range(0, share, _CHUNK):
            plan.append((j, b, dst0, share, rel, min(_CHUNK, share - rel)))
    nchunks = len(plan)

    mesh = plsc.VectorSubcoreMesh(core_axis_name="c", subcore_axis_name="s")

    @functools.partial(
        pl.kernel,
        mesh=mesh,
        out_type=(
            jax.ShapeDtypeStruct((nbins, max_len, hidden), jnp.float32),
            jax.ShapeDtypeStruct((nbins, max_len), jnp.float32),
        ),
        scratch_types=(
            [pltpu.VMEM((_CHUNK, hidden), jnp.float32)] * _NBUF
            + [
                pltpu.VMEM((16, hidden), jnp.float32),
                pltpu.VMEM((nbins, km), jnp.float32),
            ]
            + [pltpu.SemaphoreType.DMA] * (2 * _NBUF)
            + [pltpu.SemaphoreType.DMA]
        ),
    )
    def _pack(s0, s1, s2, s3, s4, s5, s6, s7, out_ref, mask_ref, *scratch):
        seq_refs = [s0, s1, s2, s3, s4, s5, s6, s7]
        bufs = list(scratch[:_NBUF])
        zbuf, mbuf = scratch[_NBUF], scratch[_NBUF + 1]
        rsems = list(scratch[_NBUF + 2:2 * _NBUF + 2])
        wsems = list(scratch[2 * _NBUF + 2:3 * _NBUF + 2])
        zsem = scratch[3 * _NBUF + 2]
        wid = lax.axis_index("s") * NC + lax.axis_index("c")

        rh = [None] * nchunks
        wh = [None] * nchunks

        def start_read(i):
            j, b, dst0, share, rel, cnt = plan[i]
            rh[i] = pltpu.async_copy(
                seq_refs[j].at[pl.ds(wid * share + rel, cnt), :],
                bufs[i % _NBUF].at[pl.ds(0, cnt)],
                rsems[i % _NBUF],
            )

        def start_write(i):
            j, b, dst0, share, rel, cnt = plan[i]
            wh[i] = pltpu.async_copy(
                bufs[i % _NBUF].at[pl.ds(0, cnt)],
                out_ref.at[b, pl.ds(dst0 + wid * share + rel, cnt), :],
                wsems[i % _NBUF],
            )

        for i in range(min(_NBUF - 1, nchunks)):
            start_read(i)

        zv = jnp.zeros((16,), jnp.float32)
        def _zrow(i, c):
            def _zcol(jj, cc):
                for u in range(4):
                    zbuf[i, pl.ds(jj * 64 + u * 16, 16)] = zv
                return cc
            return lax.fori_loop(0, hidden // 64, _zcol, c)
        lax.fori_loop(0, 16, _zrow, 0)

        aux = []
        for (b, off, p) in pads:
            kp = p // W
            base = off + wid * kp
            for c0 in range(0, kp, 16):
                aux.append(
                    pltpu.async_copy(
                        zbuf, out_ref.at[b, pl.ds(base + c0, 16), :], zsem
                    )
                )

        iot = lax.iota(jnp.int32, 16)
        col0 = wid * km
        for b in range(nbins):
            for jj in range(km // 16):
                col = col0 + jj * 16 + iot
                mbuf[b, pl.ds(jj * 16, 16)] = jnp.where(
                    col < used[b], jnp.float32(1.0), jnp.float32(0.0)
                )
            aux.append(
                pltpu.async_copy(
                    mbuf.at[pl.ds(b, 1)],
                    mask_ref.at[pl.ds(b, 1), pl.ds(col0, km)],
                    zsem,
                )
            )

        for i in range(nchunks):
            if i + _NBUF - 1 < nchunks:
                if i >= 1:
                    wh[i - 1].wait()
                start_read(i + _NBUF - 1)
            rh[i].wait()
            start_write(i)

        for i in range(max(0, nchunks - _NBUF), nchunks):
            wh[i].wait()
        for h in aux:
            h.wait()

    return _pack(*seqs)
